# baseline (speedup 1.0000x reference)
import functools

import jax
import jax.numpy as jnp
from jax import lax
from jax.experimental import pallas as pl
from jax.experimental.pallas import tpu as pltpu
from jax.experimental.pallas import tpu_sc as plsc

_B = 128
_V = 100000
_NW = 32
_RPW = _B // _NW
_NV = _V // 16
_UNROLL = 8


def _ffs16(over, lane):
    return jnp.min(jnp.where(over, lane, 16))


def _scan_hist(h_ref, budget, run0, lane):
    def it(i, carry):
        run, found, bidx, above = carry
        g = 63 - i
        v = h_ref[pl.ds(g * 16, 16)]
        rv = lax.rev(v, (0,))
        cs = plsc.cumsum(rv)
        tot = run + cs
        over = tot > budget
        ks = _ffs16(over, lane)
        newf = ks < 16
        bnd = g * 16 + 15 - ks
        csk = jnp.max(jnp.where(lane == ks, cs, -jnp.inf))
        rvk = jnp.max(jnp.where(lane == ks, rv, -jnp.inf))
        anew = run + csk - rvk
        take = jnp.logical_and(jnp.logical_not(found), newf)
        bidx = jnp.where(take, bnd, bidx)
        above = jnp.where(take, anew, above)
        found = jnp.logical_or(found, newf)
        run = run + jnp.sum(v)
        return run, found, bidx, above

    def cond(carry):
        i, *_ = carry
        _, _, found, _, _ = carry
        return jnp.logical_and(i < 64, jnp.logical_not(found))

    def body(carry):
        i, run, found, bidx, above = carry
        run, found, bidx, above = it(i, (run, found, bidx, above))
        return i + 1, run, found, bidx, above

    _, run, found, bidx, above = lax.while_loop(
        cond, body,
        (jnp.int32(0), run0, jnp.bool_(False), jnp.int32(0),
         jnp.float32(0.0)))
    return found, bidx, above


def _row_body(src, dst, x_v, h1, h2, h3, qs, lane):
    V = _V
    pltpu.sync_copy(src, x_v)

    zero16 = jnp.zeros((16,), jnp.float32)

    @plsc.parallel_loop(0, 64, unroll=_UNROLL)
    def _(i):
        h1[pl.ds(i * 16, 16)] = zero16
        h2[pl.ds(i * 16, 16)] = zero16
        h3[pl.ds(i * 16, 16)] = zero16

    @plsc.parallel_loop(0, _NV, unroll=_UNROLL)
    def _(i):
        v = x_v[pl.ds(i * 16, 16)]
        bits = plsc.bitcast(v, jnp.int32)
        plsc.addupdate_scatter(h1, [bits >> 20], v)

    f1, B1, Aab = _scan_hist(h1, qs, jnp.float32(0.0), lane)
    budget1 = qs - Aab

    b1vec = jnp.broadcast_to(B1, (16,))

    @plsc.parallel_loop(0, _NV, unroll=_UNROLL)
    def _(i):
        v = x_v[pl.ds(i * 16, 16)]
        bits = plsc.bitcast(v, jnp.int32)
        sel = (bits >> 20) == b1vec
        idx = jnp.bitwise_and(bits >> 10, 1023)
        plsc.addupdate_scatter(h2, [idx], v, mask=sel)

    f2, B2, Aab2 = _scan_hist(h2, budget1, jnp.float32(0.0), lane)
    budget2 = budget1 - Aab2

    b2vec = jnp.broadcast_to(B2, (16,))

    @plsc.parallel_loop(0, _NV, unroll=_UNROLL)
    def _(i):
        v = x_v[pl.ds(i * 16, 16)]
        bits = plsc.bitcast(v, jnp.int32)
        sel = jnp.logical_and((bits >> 20) == b1vec,
                              jnp.bitwise_and(bits >> 10, 1023) == b2vec)
        idx = jnp.bitwise_and(bits, 1023)
        plsc.addupdate_scatter(h3, [idx], v, mask=sel)

    f3, B3, _ = _scan_hist(h3, budget2, jnp.float32(0.0), lane)

    u12 = jnp.bitwise_or(B1 << 20, B2 << 10)
    tbits = jnp.where(
        f1,
        jnp.where(
            f2,
            jnp.where(f3, jnp.bitwise_or(u12, B3) + 1, u12),
            B1 << 20),
        jnp.int32(0))
    tbvec = jnp.broadcast_to(tbits, (16,))

    one16 = jnp.ones((16,), jnp.float32)

    @plsc.parallel_loop(0, _NV, unroll=_UNROLL,
                        carry=jnp.zeros((16,), jnp.bool_))
    def p4(i, acc):
        v = x_v[pl.ds(i * 16, 16)]
        bits = plsc.bitcast(v, jnp.int32)
        incl = bits >= tbvec
        x_v[pl.ds(i * 16, 16)] = jnp.where(incl, one16, zero16)
        return jnp.logical_or(acc, incl)

    none_incl = jnp.logical_not(jnp.max(p4.astype(jnp.int32)) > 0)

    @pl.when(none_incl)
    def _():
        def amx(i, carry):
            m, midx = carry
            v = x_v[pl.ds(i * 16, 16)]
            upd = v > m
            midx = jnp.where(upd, i * 16 + lane, midx)
            m = jnp.where(upd, v, m)
            return m, midx

        pltpu.sync_copy(src, x_v)
        m, midx = lax.fori_loop(
            0, _NV, amx,
            (jnp.full((16,), -1.0, jnp.float32),
             jnp.zeros((16,), jnp.int32)))
        gmax = jnp.max(m)
        amax = jnp.min(jnp.where(m == gmax, midx, jnp.int32(0x7FFFFFFF)))

        @plsc.parallel_loop(0, _NV, unroll=_UNROLL)
        def _(i):
            x_v[pl.ds(i * 16, 16)] = zero16

        plsc.store_scatter(x_v, [jnp.broadcast_to(amax, (16,))], one16,
                           mask=lane == 0)

    pltpu.sync_copy(x_v, dst)


def _sc_kernel_body(pred_hbm, qv_hbm, out_hbm, x_v, h1, h2, h3, qv_v):
    wid = lax.axis_index("s") * 2 + lax.axis_index("c")
    lane = lax.iota(jnp.int32, 16)
    pltpu.sync_copy(qv_hbm, qv_v)
    qs = jnp.max(qv_v[...])
    for rr in range(_RPW):
        r = wid * _RPW + rr
        _row_body(pred_hbm.at[r], out_hbm.at[r],
                  x_v, h1, h2, h3, qs, lane)


@jax.jit
def kernel(pred, qhat):
    b, v = pred.shape
    qv = jnp.full((16,), qhat, jnp.float32)
    mesh = plsc.VectorSubcoreMesh(core_axis_name="c", subcore_axis_name="s")
    run = pl.kernel(
        _sc_kernel_body,
        out_type=jax.ShapeDtypeStruct((b, v), jnp.float32),
        mesh=mesh,
        compiler_params=pltpu.CompilerParams(
            needs_layout_passes=False, use_tc_tiling_on_sc=False),
        scratch_types=[
            pltpu.VMEM((_V,), jnp.float32),
            pltpu.VMEM((1024,), jnp.float32),
            pltpu.VMEM((1024,), jnp.float32),
            pltpu.VMEM((1024,), jnp.float32),
            pltpu.VMEM((16,), jnp.float32),
        ],
    )
    mask = run(pred, qv)
    return jnp.concatenate([mask, pred], axis=1)


# speedup vs baseline: 1.0536x; 1.0536x over previous
import functools

import jax
import jax.numpy as jnp
from jax import lax
from jax.experimental import pallas as pl
from jax.experimental.pallas import tpu as pltpu
from jax.experimental.pallas import tpu_sc as plsc

_B = 128
_V = 100000
_NW = 32
_RPW = _B // _NW
_NV = _V // 16
_UNROLL = 8


def _ffs16(over, lane):
    return jnp.min(jnp.where(over, lane, 16))


def _scan_hist(h_ref, budget, run0, lane):
    def it(i, carry):
        run, found, bidx, above = carry
        g = 63 - i
        v = h_ref[pl.ds(g * 16, 16)]
        rv = lax.rev(v, (0,))
        cs = plsc.cumsum(rv)
        tot = run + cs
        over = tot > budget
        ks = _ffs16(over, lane)
        newf = ks < 16
        bnd = g * 16 + 15 - ks
        csk = jnp.max(jnp.where(lane == ks, cs, -jnp.inf))
        rvk = jnp.max(jnp.where(lane == ks, rv, -jnp.inf))
        anew = run + csk - rvk
        take = jnp.logical_and(jnp.logical_not(found), newf)
        bidx = jnp.where(take, bnd, bidx)
        above = jnp.where(take, anew, above)
        found = jnp.logical_or(found, newf)
        run = run + jnp.sum(v)
        return run, found, bidx, above

    def cond(carry):
        i, *_ = carry
        _, _, found, _, _ = carry
        return jnp.logical_and(i < 64, jnp.logical_not(found))

    def body(carry):
        i, run, found, bidx, above = carry
        run, found, bidx, above = it(i, (run, found, bidx, above))
        return i + 1, run, found, bidx, above

    _, run, found, bidx, above = lax.while_loop(
        cond, body,
        (jnp.int32(0), run0, jnp.bool_(False), jnp.int32(0),
         jnp.float32(0.0)))
    return found, bidx, above


def _row_body(src, dst, x_v, h1, h2, h3, qs, lane, sem):
    V = _V
    pltpu.sync_copy(src, x_v)
    pred_out = pltpu.async_copy(x_v, dst.at[pl.ds(V, V)], sem)

    zero16 = jnp.zeros((16,), jnp.float32)

    @plsc.parallel_loop(0, 64, unroll=_UNROLL)
    def _(i):
        h1[pl.ds(i * 16, 16)] = zero16
        h2[pl.ds(i * 16, 16)] = zero16
        h3[pl.ds(i * 16, 16)] = zero16

    @plsc.parallel_loop(0, _NV, unroll=_UNROLL)
    def _(i):
        v = x_v[pl.ds(i * 16, 16)]
        bits = plsc.bitcast(v, jnp.int32)
        plsc.addupdate_scatter(h1, [bits >> 20], v)

    f1, B1, Aab = _scan_hist(h1, qs, jnp.float32(0.0), lane)
    budget1 = qs - Aab

    b1vec = jnp.broadcast_to(B1, (16,))

    @plsc.parallel_loop(0, _NV, unroll=_UNROLL)
    def _(i):
        v = x_v[pl.ds(i * 16, 16)]
        bits = plsc.bitcast(v, jnp.int32)
        sel = (bits >> 20) == b1vec
        idx = jnp.bitwise_and(bits >> 10, 1023)
        plsc.addupdate_scatter(h2, [idx], v, mask=sel)

    f2, B2, Aab2 = _scan_hist(h2, budget1, jnp.float32(0.0), lane)
    budget2 = budget1 - Aab2

    b2vec = jnp.broadcast_to(B2, (16,))

    @plsc.parallel_loop(0, _NV, unroll=_UNROLL)
    def _(i):
        v = x_v[pl.ds(i * 16, 16)]
        bits = plsc.bitcast(v, jnp.int32)
        sel = jnp.logical_and((bits >> 20) == b1vec,
                              jnp.bitwise_and(bits >> 10, 1023) == b2vec)
        idx = jnp.bitwise_and(bits, 1023)
        plsc.addupdate_scatter(h3, [idx], v, mask=sel)

    f3, B3, _ = _scan_hist(h3, budget2, jnp.float32(0.0), lane)

    u12 = jnp.bitwise_or(B1 << 20, B2 << 10)
    tbits = jnp.where(
        f1,
        jnp.where(
            f2,
            jnp.where(f3, jnp.bitwise_or(u12, B3) + 1, u12),
            B1 << 20),
        jnp.int32(0))
    tbvec = jnp.broadcast_to(tbits, (16,))

    pred_out.wait()

    one16 = jnp.ones((16,), jnp.float32)

    @plsc.parallel_loop(0, _NV, unroll=_UNROLL,
                        carry=jnp.zeros((16,), jnp.bool_))
    def p4(i, acc):
        v = x_v[pl.ds(i * 16, 16)]
        bits = plsc.bitcast(v, jnp.int32)
        incl = bits >= tbvec
        x_v[pl.ds(i * 16, 16)] = jnp.where(incl, one16, zero16)
        return jnp.logical_or(acc, incl)

    none_incl = jnp.logical_not(jnp.max(p4.astype(jnp.int32)) > 0)

    @pl.when(none_incl)
    def _():
        def amx(i, carry):
            m, midx = carry
            v = x_v[pl.ds(i * 16, 16)]
            upd = v > m
            midx = jnp.where(upd, i * 16 + lane, midx)
            m = jnp.where(upd, v, m)
            return m, midx

        pltpu.sync_copy(src, x_v)
        m, midx = lax.fori_loop(
            0, _NV, amx,
            (jnp.full((16,), -1.0, jnp.float32),
             jnp.zeros((16,), jnp.int32)))
        gmax = jnp.max(m)
        amax = jnp.min(jnp.where(m == gmax, midx, jnp.int32(0x7FFFFFFF)))

        @plsc.parallel_loop(0, _NV, unroll=_UNROLL)
        def _(i):
            x_v[pl.ds(i * 16, 16)] = zero16

        plsc.store_scatter(x_v, [jnp.broadcast_to(amax, (16,))], one16,
                           mask=lane == 0)

    pltpu.sync_copy(x_v, dst.at[pl.ds(0, V)])


def _sc_kernel_body(pred_hbm, qv_hbm, out_hbm, x_v, h1, h2, h3, qv_v, sem):
    wid = lax.axis_index("s") * 2 + lax.axis_index("c")
    lane = lax.iota(jnp.int32, 16)
    pltpu.sync_copy(qv_hbm, qv_v)
    qs = jnp.max(qv_v[...])
    for rr in range(_RPW):
        r = wid * _RPW + rr
        _row_body(pred_hbm.at[r], out_hbm.at[r],
                  x_v, h1, h2, h3, qs, lane, sem)


@jax.jit
def kernel(pred, qhat):
    b, v = pred.shape
    qv = jnp.full((16,), qhat, jnp.float32)
    mesh = plsc.VectorSubcoreMesh(core_axis_name="c", subcore_axis_name="s")
    run = pl.kernel(
        _sc_kernel_body,
        out_type=jax.ShapeDtypeStruct((b, 2 * v), jnp.float32),
        mesh=mesh,
        compiler_params=pltpu.CompilerParams(
            needs_layout_passes=False, use_tc_tiling_on_sc=False),
        scratch_types=[
            pltpu.VMEM((_V,), jnp.float32),
            pltpu.VMEM((1024,), jnp.float32),
            pltpu.VMEM((1024,), jnp.float32),
            pltpu.VMEM((1024,), jnp.float32),
            pltpu.VMEM((16,), jnp.float32),
            pltpu.SemaphoreType.DMA,
        ],
    )
    return run(pred, qv)


# speedup vs baseline: 1.0696x; 1.0152x over previous
import functools

import jax
import jax.numpy as jnp
from jax import lax
from jax.experimental import pallas as pl
from jax.experimental.pallas import tpu as pltpu
from jax.experimental.pallas import tpu_sc as plsc

_B = 128
_V = 100000
_NW = 32
_RPW = _B // _NW
_NV = _V // 16
_UNROLL = 16


def _ffs16(over, lane):
    return jnp.min(jnp.where(over, lane, 16))


def _scan_hist(h_ref, budget, run0, lane):
    def it(i, carry):
        run, found, bidx, above = carry
        g = 63 - i
        v = h_ref[pl.ds(g * 16, 16)]
        rv = lax.rev(v, (0,))
        cs = plsc.cumsum(rv)
        tot = run + cs
        over = tot > budget
        ks = _ffs16(over, lane)
        newf = ks < 16
        bnd = g * 16 + 15 - ks
        csk = jnp.max(jnp.where(lane == ks, cs, -jnp.inf))
        rvk = jnp.max(jnp.where(lane == ks, rv, -jnp.inf))
        anew = run + csk - rvk
        take = jnp.logical_and(jnp.logical_not(found), newf)
        bidx = jnp.where(take, bnd, bidx)
        above = jnp.where(take, anew, above)
        found = jnp.logical_or(found, newf)
        run = run + jnp.sum(v)
        return run, found, bidx, above

    run, found, bidx, above = lax.fori_loop(
        0, 64, it, (run0, jnp.bool_(False), jnp.int32(0), jnp.float32(0.0)))
    return found, bidx, above


def _row_body(src, dst, x_v, h1, h2, h3, qs, lane, sem):
    V = _V
    pltpu.sync_copy(src, x_v)
    pred_out = pltpu.async_copy(x_v, dst.at[pl.ds(V, V)], sem)

    zero16 = jnp.zeros((16,), jnp.float32)

    @plsc.parallel_loop(0, 64, unroll=_UNROLL)
    def _(i):
        h1[pl.ds(i * 16, 16)] = zero16
        h2[pl.ds(i * 16, 16)] = zero16
        h3[pl.ds(i * 16, 16)] = zero16

    @plsc.parallel_loop(0, _NV, unroll=_UNROLL)
    def _(i):
        v = x_v[pl.ds(i * 16, 16)]
        bits = plsc.bitcast(v, jnp.int32)
        plsc.addupdate_scatter(h1, [bits >> 20], v)

    f1, B1, Aab = _scan_hist(h1, qs, jnp.float32(0.0), lane)
    budget1 = qs - Aab

    b1vec = jnp.broadcast_to(B1, (16,))

    @plsc.parallel_loop(0, _NV, unroll=_UNROLL)
    def _(i):
        v = x_v[pl.ds(i * 16, 16)]
        bits = plsc.bitcast(v, jnp.int32)
        sel = (bits >> 20) == b1vec
        idx = jnp.bitwise_and(bits >> 10, 1023)
        plsc.addupdate_scatter(h2, [idx], v, mask=sel)

    f2, B2, Aab2 = _scan_hist(h2, budget1, jnp.float32(0.0), lane)
    budget2 = budget1 - Aab2

    b2vec = jnp.broadcast_to(B2, (16,))

    @plsc.parallel_loop(0, _NV, unroll=_UNROLL)
    def _(i):
        v = x_v[pl.ds(i * 16, 16)]
        bits = plsc.bitcast(v, jnp.int32)
        sel = jnp.logical_and((bits >> 20) == b1vec,
                              jnp.bitwise_and(bits >> 10, 1023) == b2vec)
        idx = jnp.bitwise_and(bits, 1023)
        plsc.addupdate_scatter(h3, [idx], v, mask=sel)

    f3, B3, _ = _scan_hist(h3, budget2, jnp.float32(0.0), lane)

    u12 = jnp.bitwise_or(B1 << 20, B2 << 10)
    tbits = jnp.where(
        f1,
        jnp.where(
            f2,
            jnp.where(f3, jnp.bitwise_or(u12, B3) + 1, u12),
            B1 << 20),
        jnp.int32(0))
    tbvec = jnp.broadcast_to(tbits, (16,))

    pred_out.wait()

    one16 = jnp.ones((16,), jnp.float32)

    @plsc.parallel_loop(0, _NV, unroll=_UNROLL,
                        carry=jnp.zeros((16,), jnp.bool_))
    def p4(i, acc):
        v = x_v[pl.ds(i * 16, 16)]
        bits = plsc.bitcast(v, jnp.int32)
        incl = bits >= tbvec
        x_v[pl.ds(i * 16, 16)] = jnp.where(incl, one16, zero16)
        return jnp.logical_or(acc, incl)

    none_incl = jnp.logical_not(jnp.max(p4.astype(jnp.int32)) > 0)

    @pl.when(none_incl)
    def _():
        def amx(i, carry):
            m, midx = carry
            v = x_v[pl.ds(i * 16, 16)]
            upd = v > m
            midx = jnp.where(upd, i * 16 + lane, midx)
            m = jnp.where(upd, v, m)
            return m, midx

        pltpu.sync_copy(src, x_v)
        m, midx = lax.fori_loop(
            0, _NV, amx,
            (jnp.full((16,), -1.0, jnp.float32),
             jnp.zeros((16,), jnp.int32)))
        gmax = jnp.max(m)
        amax = jnp.min(jnp.where(m == gmax, midx, jnp.int32(0x7FFFFFFF)))

        @plsc.parallel_loop(0, _NV, unroll=_UNROLL)
        def _(i):
            x_v[pl.ds(i * 16, 16)] = zero16

        plsc.store_scatter(x_v, [jnp.broadcast_to(amax, (16,))], one16,
                           mask=lane == 0)

    pltpu.sync_copy(x_v, dst.at[pl.ds(0, V)])


def _sc_kernel_body(pred_hbm, qv_hbm, out_hbm, x_v, h1, h2, h3, qv_v, sem):
    wid = lax.axis_index("s") * 2 + lax.axis_index("c")
    lane = lax.iota(jnp.int32, 16)
    pltpu.sync_copy(qv_hbm, qv_v)
    qs = jnp.max(qv_v[...])
    for rr in range(_RPW):
        r = wid * _RPW + rr
        _row_body(pred_hbm.at[r], out_hbm.at[r],
                  x_v, h1, h2, h3, qs, lane, sem)


@jax.jit
def kernel(pred, qhat):
    b, v = pred.shape
    qv = jnp.full((16,), qhat, jnp.float32)
    mesh = plsc.VectorSubcoreMesh(core_axis_name="c", subcore_axis_name="s")
    run = pl.kernel(
        _sc_kernel_body,
        out_type=jax.ShapeDtypeStruct((b, 2 * v), jnp.float32),
        mesh=mesh,
        compiler_params=pltpu.CompilerParams(
            needs_layout_passes=False, use_tc_tiling_on_sc=False),
        scratch_types=[
            pltpu.VMEM((_V,), jnp.float32),
            pltpu.VMEM((1024,), jnp.float32),
            pltpu.VMEM((1024,), jnp.float32),
            pltpu.VMEM((1024,), jnp.float32),
            pltpu.VMEM((16,), jnp.float32),
            pltpu.SemaphoreType.DMA,
        ],
    )
    return run(pred, qv)


# speedup vs baseline: 1.1230x; 1.0499x over previous
import functools

import jax
import jax.numpy as jnp
from jax import lax
from jax.experimental import pallas as pl
from jax.experimental.pallas import tpu as pltpu
from jax.experimental.pallas import tpu_sc as plsc

_B = 128
_V = 100000
_NW = 32
_RPW = _B // _NW
_NV = _V // 16
_UNROLL = 8


def _ffs16(over, lane):
    return jnp.min(jnp.where(over, lane, 16))


def _scan_hist(h_ref, budget, run0, lane):
    def it(i, carry):
        run, found, bidx, above = carry
        g = 63 - i
        v = h_ref[pl.ds(g * 16, 16)]
        rv = lax.rev(v, (0,))
        cs = plsc.cumsum(rv)
        tot = run + cs
        over = tot > budget
        ks = _ffs16(over, lane)
        newf = ks < 16
        bnd = g * 16 + 15 - ks
        csk = jnp.max(jnp.where(lane == ks, cs, -jnp.inf))
        rvk = jnp.max(jnp.where(lane == ks, rv, -jnp.inf))
        anew = run + csk - rvk
        take = jnp.logical_and(jnp.logical_not(found), newf)
        bidx = jnp.where(take, bnd, bidx)
        above = jnp.where(take, anew, above)
        found = jnp.logical_or(found, newf)
        run = run + jnp.sum(v)
        return run, found, bidx, above

    run, found, bidx, above = lax.fori_loop(
        0, 64, it, (run0, jnp.bool_(False), jnp.int32(0), jnp.float32(0.0)))
    return found, bidx, above


_NCH = 5
_CW = _V // _NCH
_CNV = _CW // 16


def _row_body(src, dst, x_v, h1, h2, h3, qs, lane, sem, sem2):
    V = _V
    loads = [pltpu.async_copy(src.at[pl.ds(c * _CW, _CW)],
                              x_v.at[pl.ds(c * _CW, _CW)], sem)
             for c in range(_NCH)]
    zero16 = jnp.zeros((16,), jnp.float32)

    @plsc.parallel_loop(0, 64, unroll=_UNROLL)
    def _(i):
        h1[pl.ds(i * 16, 16)] = zero16
        h2[pl.ds(i * 16, 16)] = zero16
        h3[pl.ds(i * 16, 16)] = zero16

    for c in range(_NCH):
        loads[c].wait()

        @plsc.parallel_loop(c * _CNV, (c + 1) * _CNV, unroll=_UNROLL)
        def _(i):
            v = x_v[pl.ds(i * 16, 16)]
            bits = plsc.bitcast(v, jnp.int32)
            plsc.addupdate_scatter(h1, [bits >> 20], v)

    pred_out = pltpu.async_copy(x_v, dst.at[pl.ds(V, V)], sem)

    f1, B1, Aab = _scan_hist(h1, qs, jnp.float32(0.0), lane)
    budget1 = qs - Aab

    b1vec = jnp.broadcast_to(B1, (16,))

    @plsc.parallel_loop(0, _NV, unroll=_UNROLL)
    def _(i):
        v = x_v[pl.ds(i * 16, 16)]
        bits = plsc.bitcast(v, jnp.int32)
        sel = (bits >> 20) == b1vec
        idx = jnp.bitwise_and(bits >> 10, 1023)
        plsc.addupdate_scatter(h2, [idx], v, mask=sel)

    f2, B2, Aab2 = _scan_hist(h2, budget1, jnp.float32(0.0), lane)
    budget2 = budget1 - Aab2

    b2vec = jnp.broadcast_to(B2, (16,))

    @plsc.parallel_loop(0, _NV, unroll=_UNROLL)
    def _(i):
        v = x_v[pl.ds(i * 16, 16)]
        bits = plsc.bitcast(v, jnp.int32)
        sel = jnp.logical_and((bits >> 20) == b1vec,
                              jnp.bitwise_and(bits >> 10, 1023) == b2vec)
        idx = jnp.bitwise_and(bits, 1023)
        plsc.addupdate_scatter(h3, [idx], v, mask=sel)

    f3, B3, _ = _scan_hist(h3, budget2, jnp.float32(0.0), lane)

    u12 = jnp.bitwise_or(B1 << 20, B2 << 10)
    tbits = jnp.where(
        f1,
        jnp.where(
            f2,
            jnp.where(f3, jnp.bitwise_or(u12, B3) + 1, u12),
            B1 << 20),
        jnp.int32(0))
    tbvec = jnp.broadcast_to(tbits, (16,))

    pred_out.wait()

    one16 = jnp.ones((16,), jnp.float32)
    acc = jnp.zeros((16,), jnp.bool_)
    stores = []
    for c in range(_NCH):
        @plsc.parallel_loop(c * _CNV, (c + 1) * _CNV, unroll=_UNROLL,
                            carry=acc)
        def p4(i, a):
            v = x_v[pl.ds(i * 16, 16)]
            bits = plsc.bitcast(v, jnp.int32)
            incl = bits >= tbvec
            x_v[pl.ds(i * 16, 16)] = jnp.where(incl, one16, zero16)
            return jnp.logical_or(a, incl)

        acc = p4
        stores.append(pltpu.async_copy(
            x_v.at[pl.ds(c * _CW, _CW)], dst.at[pl.ds(c * _CW, _CW)], sem2))

    none_incl = jnp.logical_not(jnp.max(acc.astype(jnp.int32)) > 0)

    for st in stores:
        st.wait()

    @pl.when(none_incl)
    def _():
        def amx(i, carry):
            m, midx = carry
            v = x_v[pl.ds(i * 16, 16)]
            upd = v > m
            midx = jnp.where(upd, i * 16 + lane, midx)
            m = jnp.where(upd, v, m)
            return m, midx

        pltpu.sync_copy(src, x_v)
        m, midx = lax.fori_loop(
            0, _NV, amx,
            (jnp.full((16,), -1.0, jnp.float32),
             jnp.zeros((16,), jnp.int32)))
        gmax = jnp.max(m)
        amax = jnp.min(jnp.where(m == gmax, midx, jnp.int32(0x7FFFFFFF)))

        @plsc.parallel_loop(0, _NV, unroll=_UNROLL)
        def _(i):
            x_v[pl.ds(i * 16, 16)] = zero16

        plsc.store_scatter(x_v, [jnp.broadcast_to(amax, (16,))], one16,
                           mask=lane == 0)
        pltpu.sync_copy(x_v, dst.at[pl.ds(0, V)])


def _sc_kernel_body(pred_hbm, qv_hbm, out_hbm, x_v, h1, h2, h3, qv_v, sem,
                    sem2):
    wid = lax.axis_index("s") * 2 + lax.axis_index("c")
    lane = lax.iota(jnp.int32, 16)
    pltpu.sync_copy(qv_hbm, qv_v)
    qs = jnp.max(qv_v[...])
    for rr in range(_RPW):
        r = wid * _RPW + rr
        _row_body(pred_hbm.at[r], out_hbm.at[r],
                  x_v, h1, h2, h3, qs, lane, sem, sem2)


@jax.jit
def kernel(pred, qhat):
    b, v = pred.shape
    qv = jnp.full((16,), qhat, jnp.float32)
    mesh = plsc.VectorSubcoreMesh(core_axis_name="c", subcore_axis_name="s")
    run = pl.kernel(
        _sc_kernel_body,
        out_type=jax.ShapeDtypeStruct((b, 2 * v), jnp.float32),
        mesh=mesh,
        compiler_params=pltpu.CompilerParams(
            needs_layout_passes=False, use_tc_tiling_on_sc=False),
        scratch_types=[
            pltpu.VMEM((_V,), jnp.float32),
            pltpu.VMEM((1024,), jnp.float32),
            pltpu.VMEM((1024,), jnp.float32),
            pltpu.VMEM((1024,), jnp.float32),
            pltpu.VMEM((16,), jnp.float32),
            pltpu.SemaphoreType.DMA,
            pltpu.SemaphoreType.DMA,
        ],
    )
    return run(pred, qv)
